# Initial kernel scaffold; baseline (speedup 1.0000x reference)
import jax
import jax.numpy as jnp
from jax.experimental import pallas as pl


def kernel(X, edge_index, edge_weight, H, Wz, bz, Wr, br, Wh, bh, LzW, Lzb, LrW, Lrb, LhW, Lhb):
    raise NotImplementedError("write your pallas kernel here")


# speedup vs baseline: 22.4156x; 22.4156x over previous
import functools

import jax
import jax.numpy as jnp
from jax import lax
from jax.experimental import pallas as pl
from jax.experimental.pallas import tpu as pltpu
from jax.experimental.pallas import tpu_sc as plsc

N = 10000
E = 320000
F = 128
NC = 2
NS = 16
NW = NC * NS
CHUNK = 128
NCHUNK = 79
EP = NW * NCHUNK * CHUNK
NPAD = 10240
BR = 1000
BRP = 1024


def _sc_deg(col3, ew3, zeros_n):
    mesh = plsc.VectorSubcoreMesh(core_axis_name="c", subcore_axis_name="s")

    @functools.partial(
        pl.kernel,
        mesh=mesh,
        out_type=jax.ShapeDtypeStruct((NC, NPAD), jnp.float32),
        scratch_types=[
            pltpu.VMEM((NCHUNK, CHUNK), jnp.int32),
            pltpu.VMEM((NCHUNK, CHUNK), jnp.float32),
            pltpu.VMEM_SHARED((NPAD,), jnp.float32),
        ],
    )
    def k(colh, ewh, zh, degout, colv, eww, deg):
        c = lax.axis_index("c")
        s = lax.axis_index("s")
        w = c * NS + s
        sl = pl.ds(s * (NPAD // NS), NPAD // NS)
        pltpu.sync_copy(zh.at[sl], deg.at[sl])
        plsc.subcore_barrier()
        pltpu.sync_copy(colh.at[w], colv)
        pltpu.sync_copy(ewh.at[w], eww)

        def chunk(j, carry):
            pltpu.sync_copy(eww.at[j], deg.at[colv.at[j]], add=True)
            return carry

        lax.fori_loop(0, NCHUNK, chunk, 0)
        plsc.subcore_barrier()
        pltpu.sync_copy(deg.at[sl], degout.at[c, sl])

    return k(col3, ew3, zeros_n)


def _xs_body(x_ref, d0_ref, d1_ref, o_ref):
    ds = lax.rsqrt(d0_ref[...] + d1_ref[...] + 1.0)
    o_ref[...] = x_ref[...] * ds


def _tc_xs(x2, d0, d1):
    return pl.pallas_call(
        _xs_body,
        grid=(NPAD // BRP,),
        in_specs=[
            pl.BlockSpec((BRP, F), lambda i: (i, 0)),
            pl.BlockSpec((BRP, 1), lambda i: (i, 0)),
            pl.BlockSpec((BRP, 1), lambda i: (i, 0)),
        ],
        out_specs=pl.BlockSpec((BRP, F), lambda i: (i, 0)),
        out_shape=jax.ShapeDtypeStruct((NPAD, F), jnp.float32),
    )(x2, d0, d1)


def _sc_agg(row3, col3, ew3, xs, zeros_nf):
    mesh = plsc.VectorSubcoreMesh(core_axis_name="c", subcore_axis_name="s")

    @functools.partial(
        pl.kernel,
        mesh=mesh,
        out_type=jax.ShapeDtypeStruct((NC, NPAD, F), jnp.float32),
        scratch_types=[
            pltpu.VMEM((NCHUNK, CHUNK), jnp.int32),
            pltpu.VMEM((NCHUNK, CHUNK), jnp.int32),
            pltpu.VMEM((NCHUNK, CHUNK), jnp.float32),
            pltpu.VMEM((CHUNK, F), jnp.float32),
            pltpu.SemaphoreType.DMA,
            pltpu.VMEM_SHARED((NPAD, F), jnp.float32),
        ],
    )
    def k(rowh, colh, ewh, xsh, zh, aggout, rowv, colv, eww, rows, sem, agg):
        c = lax.axis_index("c")
        s = lax.axis_index("s")
        w = c * NS + s
        sl = pl.ds(s * (NPAD // NS), NPAD // NS)

        @pl.when(c == 0)
        def _():
            pltpu.sync_copy(xsh.at[sl], agg.at[sl])

        @pl.when(c != 0)
        def _():
            pltpu.sync_copy(zh.at[sl], agg.at[sl])

        plsc.subcore_barrier()
        pltpu.sync_copy(rowh.at[w], rowv)
        pltpu.sync_copy(colh.at[w], colv)
        pltpu.sync_copy(ewh.at[w], eww)

        def chunk(j, carry):
            pltpu.async_copy(xsh.at[rowv.at[j]], rows, sem).wait()

            dnums = lax.GatherDimensionNumbers(
                offset_dims=(), collapsed_slice_dims=(0,),
                start_index_map=(0,))

            def grp(g, c2):
                wgrp = eww[j, pl.ds(g * 16, 16)]
                for l in range(16):
                    wv = lax.gather(
                        wgrp, jnp.full((16, 1), l, jnp.int32), dnums, (1,),
                        mode=lax.GatherScatterMode.PROMISE_IN_BOUNDS)
                    kk = g * 16 + l
                    for t in range(F // 16):
                        rows[kk, pl.ds(t * 16, 16)] = (
                            rows[kk, pl.ds(t * 16, 16)] * wv)
                return c2

            lax.fori_loop(0, CHUNK // 16, grp, 0)
            pltpu.sync_copy(rows, agg.at[colv.at[j]], add=True)
            return carry

        lax.fori_loop(0, NCHUNK, chunk, 0)
        plsc.subcore_barrier()
        pltpu.sync_copy(agg.at[sl], aggout.at[c, sl])

    return k(row3, col3, ew3, xs, zeros_nf)


def _gru_body(a0_ref, a1_ref, h_ref, d0_ref, d1_ref,
              wz_ref, wr_ref, wh_ref, lz_ref, lr_ref, lh_ref,
              bz_ref, br_ref, bh_ref, o_ref):
    f32 = jnp.float32
    ds = lax.rsqrt(d0_ref[...] + d1_ref[...] + 1.0)
    agg = (a0_ref[...] + a1_ref[...]) * ds
    h = h_ref[...]

    def gate(w_ref, l_ref, b_ref, hv):
        lt = l_ref[0:F, :]
        lb = l_ref[F:2 * F, :]
        a = jnp.dot(w_ref[...], lt, preferred_element_type=f32)
        pre = (jnp.dot(agg, a, preferred_element_type=f32)
               + jnp.dot(hv, lb, preferred_element_type=f32)
               + b_ref[...])
        return pre

    z = jax.nn.sigmoid(gate(wz_ref, lz_ref, bz_ref, h))
    r = jax.nn.sigmoid(gate(wr_ref, lr_ref, br_ref, h))
    ht = jnp.tanh(gate(wh_ref, lh_ref, bh_ref, h * r))
    o_ref[...] = z * h + (1.0 - z) * ht


def _tc_gru(a0, a1, h2, d0, d1, wz, wr, wh, lz, lr, lh, bz2, br2, bh2):
    row_spec = pl.BlockSpec((BR, F), lambda i: (i, 0))
    col_spec = pl.BlockSpec((BR, 1), lambda i: (i, 0))
    w_spec = pl.BlockSpec((F, F), lambda i: (0, 0))
    l_spec = pl.BlockSpec((2 * F, F), lambda i: (0, 0))
    b_spec = pl.BlockSpec((1, F), lambda i: (0, 0))
    return pl.pallas_call(
        _gru_body,
        grid=(N // BR,),
        in_specs=[row_spec, row_spec, row_spec, col_spec, col_spec,
                  w_spec, w_spec, w_spec, l_spec, l_spec, l_spec,
                  b_spec, b_spec, b_spec],
        out_specs=row_spec,
        out_shape=jax.ShapeDtypeStruct((N, F), jnp.float32),
    )(a0, a1, h2, d0, d1, wz, wr, wh, lz, lr, lh, bz2, br2, bh2)


def kernel(X, edge_index, edge_weight, H, Wz, bz, Wr, br, Wh, bh,
           LzW, Lzb, LrW, Lrb, LhW, Lhb):
    x2 = X.reshape(N, F)
    h2 = H.reshape(N, F)
    ei = edge_index.astype(jnp.int32)
    pad = EP - E
    row3 = jnp.pad(ei[0], (0, pad)).reshape(NW, NCHUNK, CHUNK)
    col3 = jnp.pad(ei[1], (0, pad)).reshape(NW, NCHUNK, CHUNK)
    ew3 = jnp.pad(edge_weight, (0, pad)).reshape(NW, NCHUNK, CHUNK)
    zeros_n = jnp.zeros((NPAD,), jnp.float32)
    zeros_nf = jnp.zeros((NPAD, F), jnp.float32)

    degp = _sc_deg(col3, ew3, zeros_n)
    d0 = degp[0].reshape(NPAD, 1)
    d1 = degp[1].reshape(NPAD, 1)
    x2p = jnp.pad(x2, ((0, NPAD - N), (0, 0)))
    xs = _tc_xs(x2p, d0, d1)
    aggp = _sc_agg(row3, col3, ew3, xs, zeros_nf)

    bz2 = (bz @ LzW[:F] + Lzb).reshape(1, F)
    br2 = (br @ LrW[:F] + Lrb).reshape(1, F)
    bh2 = (bh @ LhW[:F] + Lhb).reshape(1, F)

    out = _tc_gru(aggp[0], aggp[1], h2, d0, d1,
                  Wz, Wr, Wh, LzW, LrW, LhW, bz2, br2, bh2)
    return out.reshape(1, N, F)
